# baseline (speedup 1.0000x reference)
import functools

import jax
import jax.numpy as jnp
from jax import lax
from jax.experimental import pallas as pl
from jax.experimental.pallas import tpu as pltpu
from jax.experimental.pallas import tpu_sc as plsc

N, D, K = 8192, 256, 1024
ITERS = 5
BM = 512
NBLK = N // BM
NW = 32
RPW = N // NW
QTR = RPW * D // 4
QCH = QTR // 128
ACC = K * D + K


def _rowsq_body(x_ref, out_ref):
    x = x_ref[...]
    out_ref[...] = jnp.sum(x * x, axis=1, keepdims=True)


def _rowsq(x):
    rows = x.shape[0]
    return pl.pallas_call(
        _rowsq_body,
        grid=(rows // BM,),
        in_specs=[pl.BlockSpec((BM, D), lambda i: (i, 0))],
        out_specs=pl.BlockSpec((BM, 1), lambda i: (i, 0)),
        out_shape=jax.ShapeDtypeStruct((rows, 1), jnp.float32),
    )(x)


def _fused_common(x_ref, x2_ref, s0_ref, s1_ref, n0_ref, n1_ref, c_ref,
                  cnew_ref, c2_scr):
    i = pl.program_id(0)

    @pl.when(i == 0)
    def _():
        sums = s0_ref[...] + s1_ref[...]
        counts = (n0_ref[0, :] + n1_ref[0, :])[:, None]
        new_c = sums / jnp.maximum(counts, 1.0)
        new_c = jnp.where(counts > 0, new_c, c_ref[...])
        cnew_ref[...] = new_c
        c2_scr[...] = jnp.sum(new_c * new_c, axis=1)[None, :]

    x = x_ref[...]
    c = cnew_ref[...]
    d2 = x2_ref[...] - 2.0 * jnp.dot(x, c.T) + c2_scr[...]
    return jnp.argmin(d2, axis=1).astype(jnp.int32)


def _fused_body(x_ref, x2_ref, s0_ref, s1_ref, n0_ref, n1_ref, c_ref,
                cnew_ref, idx_ref, cidx_ref, c2_scr):
    lbl = _fused_common(x_ref, x2_ref, s0_ref, s1_ref, n0_ref, n1_ref,
                        c_ref, cnew_ref, c2_scr)
    idx_ref[...] = lbl[:, None] * D + jax.lax.broadcasted_iota(
        jnp.int32, (BM, D), 1)
    cidx_ref[0, 0, :] = K * D + lbl


def _fused_final_body(x_ref, x2_ref, s0_ref, s1_ref, n0_ref, n1_ref, c_ref,
                      cnew_ref, labels_ref, c2_scr):
    lbl = _fused_common(x_ref, x2_ref, s0_ref, s1_ref, n0_ref, n1_ref,
                        c_ref, cnew_ref, c2_scr)
    labels_ref[0, 0, :] = lbl


_FUSED_IN_SPECS = [
    pl.BlockSpec((BM, D), lambda i: (i, 0)),
    pl.BlockSpec((BM, 1), lambda i: (i, 0)),
    pl.BlockSpec((K, D), lambda i: (0, 0)),
    pl.BlockSpec((K, D), lambda i: (0, 0)),
    pl.BlockSpec((1, K), lambda i: (0, 0)),
    pl.BlockSpec((1, K), lambda i: (0, 0)),
    pl.BlockSpec((K, D), lambda i: (0, 0)),
]


def _fused(x, x2, s0, s1, n0, n1, c):
    return pl.pallas_call(
        _fused_body,
        grid=(NBLK,),
        in_specs=_FUSED_IN_SPECS,
        out_specs=[
            pl.BlockSpec((K, D), lambda i: (0, 0)),
            pl.BlockSpec((BM, D), lambda i: (i, 0)),
            pl.BlockSpec((1, 1, BM), lambda i: (i, 0, 0)),
        ],
        out_shape=[
            jax.ShapeDtypeStruct((K, D), jnp.float32),
            jax.ShapeDtypeStruct((N, D), jnp.int32),
            jax.ShapeDtypeStruct((NBLK, 1, BM), jnp.int32),
        ],
        scratch_shapes=[pltpu.VMEM((1, K), jnp.float32)],
    )(x, x2, s0, s1, n0, n1, c)


def _fused_final(x, x2, s0, s1, n0, n1, c):
    return pl.pallas_call(
        _fused_final_body,
        grid=(NBLK,),
        in_specs=_FUSED_IN_SPECS,
        out_specs=[
            pl.BlockSpec((K, D), lambda i: (0, 0)),
            pl.BlockSpec((1, 1, BM), lambda i: (i, 0, 0)),
        ],
        out_shape=[
            jax.ShapeDtypeStruct((K, D), jnp.float32),
            jax.ShapeDtypeStruct((NBLK, 1, BM), jnp.int32),
        ],
        scratch_shapes=[pltpu.VMEM((1, K), jnp.float32)],
    )(x, x2, s0, s1, n0, n1, c)


def _segsum_sc_body(x_hbm, idx_hbm, cidx_hbm, zeros_hbm, ones_hbm,
                    s0_hbm, n0_hbm, s1_hbm, n1_hbm,
                    x_v0, x_v1, idx_v0, idx_v1, cidx_v, ones_v, acc_s,
                    sem_stage, sem_scat):
    x_v = (x_v0, x_v1)
    idx_v = (idx_v0, idx_v1)
    ci = lax.axis_index("c")
    si = lax.axis_index("s")
    w = si * 2 + ci
    base = w * RPW * D

    def _stage(q, buf):
        pltpu.async_copy(x_hbm.at[pl.ds(base + q * QTR, QTR)],
                         x_v[buf], sem_stage)
        pltpu.async_copy(idx_hbm.at[pl.ds(base + q * QTR, QTR)],
                         idx_v[buf], sem_stage)

    pltpu.async_copy(zeros_hbm.at[pl.ds(si * 16384, 16384)],
                     acc_s.at[pl.ds(si * 16384, 16384)], sem_stage)
    _stage(0, 0)
    pltpu.sync_copy(cidx_hbm.at[pl.ds(w * 2, 2)], cidx_v)
    pltpu.sync_copy(ones_hbm, ones_v)

    @pl.when(si < 8)
    def _():
        pltpu.sync_copy(zeros_hbm.at[pl.ds(si * 128, 128)],
                        acc_s.at[pl.ds(K * D + si * 128, 128)])

    pltpu.make_async_copy(x_hbm.at[pl.ds(0, 16384)],
                          acc_s.at[pl.ds(0, 16384)], sem_stage).wait()
    pltpu.make_async_copy(x_hbm.at[pl.ds(0, QTR)], x_v[0],
                          sem_stage).wait()
    pltpu.make_async_copy(idx_hbm.at[pl.ds(0, QTR)], idx_v[0],
                          sem_stage).wait()
    plsc.subcore_barrier()

    for q in range(4):
        buf = q % 2
        if q < 3:
            _stage(q + 1, 1 - buf)
        pltpu.async_copy(x_v[buf], acc_s.at[idx_v[buf]], sem_scat,
                         add=True)
        pltpu.make_async_copy(x_hbm.at[pl.ds(0, QTR)], x_v[buf],
                              sem_scat).wait()
        if q < 3:
            pltpu.make_async_copy(x_hbm.at[pl.ds(0, QTR)], x_v[1 - buf],
                                  sem_stage).wait()
            pltpu.make_async_copy(idx_hbm.at[pl.ds(0, QTR)],
                                  idx_v[1 - buf], sem_stage).wait()

    pltpu.async_copy(ones_v, acc_s.at[cidx_v.at[0]], sem_scat, add=True)
    pltpu.async_copy(ones_v, acc_s.at[cidx_v.at[1]], sem_scat, add=True)
    pltpu.make_async_copy(x_hbm.at[pl.ds(0, 128)], ones_v, sem_scat).wait()
    pltpu.make_async_copy(x_hbm.at[pl.ds(0, 128)], ones_v, sem_scat).wait()
    plsc.subcore_barrier()

    def _copy_out(s_hbm, n_hbm):
        pltpu.sync_copy(acc_s.at[pl.ds(si * 16384, 16384)],
                        s_hbm.at[pl.ds(si * 16384, 16384)])

        @pl.when(si < 8)
        def _():
            pltpu.sync_copy(acc_s.at[pl.ds(K * D + si * 128, 128)],
                            n_hbm.at[pl.ds(si * 128, 128)])

    @pl.when(ci == 0)
    def _():
        _copy_out(s0_hbm, n0_hbm)

    @pl.when(ci == 1)
    def _():
        _copy_out(s1_hbm, n1_hbm)


_segsum_sc = functools.partial(
    pl.kernel,
    out_type=[jax.ShapeDtypeStruct((K * D,), jnp.float32),
              jax.ShapeDtypeStruct((K,), jnp.float32),
              jax.ShapeDtypeStruct((K * D,), jnp.float32),
              jax.ShapeDtypeStruct((K,), jnp.float32)],
    mesh=plsc.VectorSubcoreMesh(core_axis_name="c", subcore_axis_name="s"),
    scratch_types=[
        pltpu.VMEM((QTR,), jnp.float32),
        pltpu.VMEM((QTR,), jnp.float32),
        pltpu.VMEM((QTR,), jnp.int32),
        pltpu.VMEM((QTR,), jnp.int32),
        pltpu.VMEM((2, 128), jnp.int32),
        pltpu.VMEM((128,), jnp.float32),
        pltpu.VMEM_SHARED((ACC,), jnp.float32),
        pltpu.SemaphoreType.DMA,
        pltpu.SemaphoreType.DMA,
    ],
)(_segsum_sc_body)


def kernel(x):
    x = x.reshape(x.shape[0], -1)
    x1d = x.reshape(-1)
    zeros = jnp.zeros((K * D,), jnp.float32)
    ones = jnp.ones((128,), jnp.float32)
    x2 = _rowsq(x)
    c = x[:K]
    s0 = s1 = zeros.reshape(K, D)
    n0 = n1 = jnp.zeros((1, K), jnp.float32)
    for _ in range(ITERS):
        c, idx, cidx = _fused(x, x2, s0, s1, n0, n1, c)
        p0s, p0n, p1s, p1n = _segsum_sc(
            x1d, idx.reshape(N * D), cidx.reshape(NW * 2, 128),
            zeros, ones)
        s0, s1 = p0s.reshape(K, D), p1s.reshape(K, D)
        n0, n1 = p0n.reshape(1, K), p1n.reshape(1, K)
    _, labels = _fused_final(x, x2, s0, s1, n0, n1, c)
    return labels.reshape(N)


# speedup vs baseline: 1.0530x; 1.0530x over previous
import functools

import jax
import jax.numpy as jnp
from jax import lax
from jax.experimental import pallas as pl
from jax.experimental.pallas import tpu as pltpu
from jax.experimental.pallas import tpu_sc as plsc

N, D, K = 8192, 256, 1024
ITERS = 5
BM = 1024
NBLK = N // BM
NW = 32
RPW = N // NW
QTR = RPW * D // 4
QCH = QTR // 128
ACC = K * D + K


def _rowsq_body(x_ref, out_ref):
    x = x_ref[...]
    out_ref[...] = jnp.sum(x * x, axis=1, keepdims=True)


def _rowsq(x):
    rows = x.shape[0]
    return pl.pallas_call(
        _rowsq_body,
        grid=(rows // BM,),
        in_specs=[pl.BlockSpec((BM, D), lambda i: (i, 0))],
        out_specs=pl.BlockSpec((BM, 1), lambda i: (i, 0)),
        out_shape=jax.ShapeDtypeStruct((rows, 1), jnp.float32),
    )(x)


def _fused_common(x_ref, x2_ref, s0_ref, s1_ref, n0_ref, n1_ref, c_ref,
                  cnew_ref, c2_scr):
    i = pl.program_id(0)

    @pl.when(i == 0)
    def _():
        sums = s0_ref[...] + s1_ref[...]
        counts = (n0_ref[0, :] + n1_ref[0, :])[:, None]
        new_c = sums / jnp.maximum(counts, 1.0)
        new_c = jnp.where(counts > 0, new_c, c_ref[...])
        cnew_ref[...] = new_c
        c2_scr[...] = jnp.sum(new_c * new_c, axis=1)[None, :]

    x = x_ref[...]
    c = cnew_ref[...]
    d2 = x2_ref[...] - 2.0 * jnp.dot(x, c.T) + c2_scr[...]
    return jnp.argmin(d2, axis=1).astype(jnp.int32)


def _fused_body(x_ref, x2_ref, s0_ref, s1_ref, n0_ref, n1_ref, c_ref,
                cnew_ref, idx_ref, cidx_ref, c2_scr):
    lbl = _fused_common(x_ref, x2_ref, s0_ref, s1_ref, n0_ref, n1_ref,
                        c_ref, cnew_ref, c2_scr)
    idx_ref[...] = lbl[:, None] * D + jax.lax.broadcasted_iota(
        jnp.int32, (BM, D), 1)
    cidx_ref[0, 0, :] = K * D + lbl


def _fused_final_body(x_ref, x2_ref, s0_ref, s1_ref, n0_ref, n1_ref, c_ref,
                      cnew_ref, labels_ref, c2_scr):
    lbl = _fused_common(x_ref, x2_ref, s0_ref, s1_ref, n0_ref, n1_ref,
                        c_ref, cnew_ref, c2_scr)
    labels_ref[0, 0, :] = lbl


_FUSED_IN_SPECS = [
    pl.BlockSpec((BM, D), lambda i: (i, 0)),
    pl.BlockSpec((BM, 1), lambda i: (i, 0)),
    pl.BlockSpec((K, D), lambda i: (0, 0)),
    pl.BlockSpec((K, D), lambda i: (0, 0)),
    pl.BlockSpec((1, K), lambda i: (0, 0)),
    pl.BlockSpec((1, K), lambda i: (0, 0)),
    pl.BlockSpec((K, D), lambda i: (0, 0)),
]


def _fused(x, x2, s0, s1, n0, n1, c):
    return pl.pallas_call(
        _fused_body,
        grid=(NBLK,),
        in_specs=_FUSED_IN_SPECS,
        out_specs=[
            pl.BlockSpec((K, D), lambda i: (0, 0)),
            pl.BlockSpec((BM, D), lambda i: (i, 0)),
            pl.BlockSpec((1, 1, BM), lambda i: (i, 0, 0)),
        ],
        out_shape=[
            jax.ShapeDtypeStruct((K, D), jnp.float32),
            jax.ShapeDtypeStruct((N, D), jnp.int32),
            jax.ShapeDtypeStruct((NBLK, 1, BM), jnp.int32),
        ],
        scratch_shapes=[pltpu.VMEM((1, K), jnp.float32)],
    )(x, x2, s0, s1, n0, n1, c)


def _fused_final(x, x2, s0, s1, n0, n1, c):
    return pl.pallas_call(
        _fused_final_body,
        grid=(NBLK,),
        in_specs=_FUSED_IN_SPECS,
        out_specs=[
            pl.BlockSpec((K, D), lambda i: (0, 0)),
            pl.BlockSpec((1, 1, BM), lambda i: (i, 0, 0)),
        ],
        out_shape=[
            jax.ShapeDtypeStruct((K, D), jnp.float32),
            jax.ShapeDtypeStruct((NBLK, 1, BM), jnp.int32),
        ],
        scratch_shapes=[pltpu.VMEM((1, K), jnp.float32)],
    )(x, x2, s0, s1, n0, n1, c)


def _segsum_sc_body(x_hbm, idx_hbm, cidx_hbm, zeros_hbm, ones_hbm,
                    s0_hbm, n0_hbm, s1_hbm, n1_hbm,
                    x_v0, x_v1, idx_v0, idx_v1, cidx_v, ones_v, acc_s,
                    sem_stage, sem_scat):
    x_v = (x_v0, x_v1)
    idx_v = (idx_v0, idx_v1)
    ci = lax.axis_index("c")
    si = lax.axis_index("s")
    w = si * 2 + ci
    base = w * RPW * D

    def _stage(q, buf):
        pltpu.async_copy(x_hbm.at[pl.ds(base + q * QTR, QTR)],
                         x_v[buf], sem_stage)
        pltpu.async_copy(idx_hbm.at[pl.ds(base + q * QTR, QTR)],
                         idx_v[buf], sem_stage)

    pltpu.async_copy(zeros_hbm.at[pl.ds(si * 16384, 16384)],
                     acc_s.at[pl.ds(si * 16384, 16384)], sem_stage)
    _stage(0, 0)
    pltpu.sync_copy(cidx_hbm.at[pl.ds(w * 2, 2)], cidx_v)
    pltpu.sync_copy(ones_hbm, ones_v)

    @pl.when(si < 8)
    def _():
        pltpu.sync_copy(zeros_hbm.at[pl.ds(si * 128, 128)],
                        acc_s.at[pl.ds(K * D + si * 128, 128)])

    pltpu.make_async_copy(x_hbm.at[pl.ds(0, 16384)],
                          acc_s.at[pl.ds(0, 16384)], sem_stage).wait()
    pltpu.make_async_copy(x_hbm.at[pl.ds(0, QTR)], x_v[0],
                          sem_stage).wait()
    pltpu.make_async_copy(idx_hbm.at[pl.ds(0, QTR)], idx_v[0],
                          sem_stage).wait()
    plsc.subcore_barrier()

    for q in range(4):
        buf = q % 2
        if q < 3:
            _stage(q + 1, 1 - buf)
        pltpu.async_copy(x_v[buf], acc_s.at[idx_v[buf]], sem_scat,
                         add=True)
        pltpu.make_async_copy(x_hbm.at[pl.ds(0, QTR)], x_v[buf],
                              sem_scat).wait()
        if q < 3:
            pltpu.make_async_copy(x_hbm.at[pl.ds(0, QTR)], x_v[1 - buf],
                                  sem_stage).wait()
            pltpu.make_async_copy(idx_hbm.at[pl.ds(0, QTR)],
                                  idx_v[1 - buf], sem_stage).wait()

    pltpu.async_copy(ones_v, acc_s.at[cidx_v.at[0]], sem_scat, add=True)
    pltpu.async_copy(ones_v, acc_s.at[cidx_v.at[1]], sem_scat, add=True)
    pltpu.make_async_copy(x_hbm.at[pl.ds(0, 128)], ones_v, sem_scat).wait()
    pltpu.make_async_copy(x_hbm.at[pl.ds(0, 128)], ones_v, sem_scat).wait()
    plsc.subcore_barrier()

    def _copy_out(s_hbm, n_hbm):
        pltpu.sync_copy(acc_s.at[pl.ds(si * 16384, 16384)],
                        s_hbm.at[pl.ds(si * 16384, 16384)])

        @pl.when(si < 8)
        def _():
            pltpu.sync_copy(acc_s.at[pl.ds(K * D + si * 128, 128)],
                            n_hbm.at[pl.ds(si * 128, 128)])

    @pl.when(ci == 0)
    def _():
        _copy_out(s0_hbm, n0_hbm)

    @pl.when(ci == 1)
    def _():
        _copy_out(s1_hbm, n1_hbm)


_segsum_sc = functools.partial(
    pl.kernel,
    out_type=[jax.ShapeDtypeStruct((K * D,), jnp.float32),
              jax.ShapeDtypeStruct((K,), jnp.float32),
              jax.ShapeDtypeStruct((K * D,), jnp.float32),
              jax.ShapeDtypeStruct((K,), jnp.float32)],
    mesh=plsc.VectorSubcoreMesh(core_axis_name="c", subcore_axis_name="s"),
    scratch_types=[
        pltpu.VMEM((QTR,), jnp.float32),
        pltpu.VMEM((QTR,), jnp.float32),
        pltpu.VMEM((QTR,), jnp.int32),
        pltpu.VMEM((QTR,), jnp.int32),
        pltpu.VMEM((2, 128), jnp.int32),
        pltpu.VMEM((128,), jnp.float32),
        pltpu.VMEM_SHARED((ACC,), jnp.float32),
        pltpu.SemaphoreType.DMA,
        pltpu.SemaphoreType.DMA,
    ],
)(_segsum_sc_body)


def kernel(x):
    x = x.reshape(x.shape[0], -1)
    x1d = x.reshape(-1)
    zeros = jnp.zeros((K * D,), jnp.float32)
    ones = jnp.ones((128,), jnp.float32)
    x2 = _rowsq(x)
    c = x[:K]
    s0 = s1 = zeros.reshape(K, D)
    n0 = n1 = jnp.zeros((1, K), jnp.float32)
    for _ in range(ITERS):
        c, idx, cidx = _fused(x, x2, s0, s1, n0, n1, c)
        p0s, p0n, p1s, p1n = _segsum_sc(
            x1d, idx.reshape(N * D), cidx.reshape(NW * 2, 128),
            zeros, ones)
        s0, s1 = p0s.reshape(K, D), p1s.reshape(K, D)
        n0, n1 = p0n.reshape(1, K), p1n.reshape(1, K)
    _, labels = _fused_final(x, x2, s0, s1, n0, n1, c)
    return labels.reshape(N)
